# baseline (speedup 1.0000x reference)
import functools

import numpy as np
import jax
from jax import lax
import jax.numpy as jnp
from jax.experimental import pallas as pl
from jax.experimental.pallas import tpu as pltpu
from jax.experimental.pallas import tpu_sc as plsc

_NUM_USERS = 10000
_NUM_ITEMS = 50000
_D = 128
_NU = 256
_T = 4
_THRESHOLD = 0.5
_IPU = 16

_NW = 32
_NI = _T * _IPU * _NU
_NUALL = _T * _NU
_IPT = _NI // _NW
_UPT = _NUALL // _NW
_ICH = 128


def _build_consts():
    rng = np.random.default_rng(42)
    user_pos = [np.unique(rng.integers(0, _NUM_ITEMS, _IPU))
                for _ in range(_NUM_USERS)]
    srng = np.random.default_rng(1234)
    su_list, idx_list, pads, p_list = [], [], [], []
    for t in range(_T):
        sampled = np.sort(srng.choice(_NUM_USERS, size=_NU, replace=False))
        items = [user_pos[int(u)] for u in sampled]
        ipad = np.zeros((_NU, _IPU), np.int32)
        for i, its in enumerate(items):
            ipad[i, :len(its)] = its
            for k in range(len(its), _IPU):
                pads.append((t, k, i))
        union = np.unique(np.concatenate(items))
        M = np.zeros((_NU, len(union)), np.float32)
        for i, its in enumerate(items):
            M[i, np.searchsorted(union, its)] = 1.0
        su_list.append(sampled.astype(np.int32))
        idx_list.append(ipad.reshape(-1))
        p_list.append(M @ M.T)
    su_all = np.concatenate(su_list).reshape(1, _T * _NU)
    idx_all = np.concatenate(idx_list).reshape(1, _T * _IPU * _NU)
    p_all = np.stack(p_list).astype(np.float32)
    return su_all, idx_all, pads, p_all


_SU_NP, _IDX_NP, _PADS, _P_NP = _build_consts()


def _sc_gather(user_emb, item_emb, su, idx, dst, zeros):
    mesh = plsc.VectorSubcoreMesh(core_axis_name="c", subcore_axis_name="s")
    n_ch = _IPT // _ICH

    @functools.partial(
        pl.kernel,
        out_type=(jax.ShapeDtypeStruct((_NUALL, _D), jnp.float32),
                  jax.ShapeDtypeStruct((_NUALL, _D), jnp.float32)),
        mesh=mesh,
        scratch_types=[
            pltpu.VMEM((_UPT,), jnp.int32),
            pltpu.VMEM((n_ch, _ICH), jnp.int32),
            pltpu.VMEM((n_ch, _ICH), jnp.int32),
            pltpu.VMEM((_UPT, _D), jnp.float32),
            pltpu.VMEM((_IPT, _D), jnp.float32),
            pltpu.VMEM_SHARED((16 * _UPT, _D), jnp.float32),
            pltpu.SemaphoreType.DMA,
        ])
    def gather_kernel(ue_hbm, ie_hbm, su_hbm, ix_hbm, dst_hbm, z_hbm,
                      ou_hbm, ob_hbm,
                      su_v, ix_v, dst_v, ur_v, ir_v, b_sh, sem):
        sidx = lax.axis_index("s")
        wid = sidx * 2 + lax.axis_index("c")
        ub = wid * _UPT
        pltpu.sync_copy(su_hbm.at[wid], su_v)
        pltpu.sync_copy(ix_hbm.at[wid], ix_v)
        pltpu.sync_copy(dst_hbm.at[sidx], dst_v)
        copies = [pltpu.async_copy(ue_hbm.at[su_v], ur_v, sem)]
        for c in range(n_ch):
            copies.append(pltpu.async_copy(
                ie_hbm.at[ix_v.at[c]], ir_v.at[pl.ds(c * _ICH, _ICH)], sem))
        pltpu.sync_copy(z_hbm, b_sh.at[pl.ds(sidx * _UPT, _UPT)])
        for cp in copies:
            cp.wait()
        pltpu.sync_copy(ur_v, ou_hbm.at[pl.ds(ub, _UPT)])
        for c in range(n_ch):
            pltpu.sync_copy(
                ir_v.at[pl.ds(c * _ICH, _ICH)], b_sh.at[dst_v.at[c]],
                add=True)
        pltpu.sync_copy(b_sh.at[pl.ds(sidx * _UPT, _UPT)],
                        ob_hbm.at[pl.ds(ub, _UPT)])

    return gather_kernel(user_emb, item_emb,
                         su.reshape(_NW, _UPT),
                         idx.reshape(_NW, n_ch, _ICH),
                         dst, zeros)


def _dense_body(u_ref, b_ref, p_ref, it0_ref, w_ref, o_ref):
    w = w_ref[...]
    it0 = jnp.broadcast_to(it0_ref[...], (_NU, _D))
    acc = jnp.zeros((_D, _D), jnp.float32)
    for t in range(_T):
        ut = u_ref[t]
        n2 = jnp.sum(ut * ut, axis=1, keepdims=True)
        un = ut / jnp.maximum(jnp.sqrt(n2), 1e-12)
        s = jnp.dot(un, un.T, preferred_element_type=jnp.float32)
        ii = jax.lax.broadcasted_iota(jnp.int32, (_NU, _NU), 0)
        jj = jax.lax.broadcasted_iota(jnp.int32, (_NU, _NU), 1)
        a = jnp.where((s > _THRESHOLD) & (ii != jj), 1.0, 0.0)
        b = b_ref[t]
        rr = jax.lax.broadcasted_iota(jnp.int32, (_NU, _D), 0)
        for (tt, _kk, uu) in _PADS:
            if tt == t:
                b = jnp.where(rr == uu, b - it0, b)
        hu = jnp.dot(a, ut, preferred_element_type=jnp.float32) + b
        pu = jnp.dot(p_ref[t], ut, preferred_element_type=jnp.float32)
        g = (jnp.dot(hu.T, hu, preferred_element_type=jnp.float32)
             + jnp.dot(ut.T, pu, preferred_element_type=jnp.float32))
        wg = jnp.dot(w, g, preferred_element_type=jnp.float32)
        wt = jnp.dot(wg, w.T, preferred_element_type=jnp.float32)
        fro = jnp.sqrt(jnp.sum(wt * wt)) + 1e-8
        acc = acc + wt / fro
    o_ref[...] = acc * (1.0 / _T)


def kernel(user_emb, item_emb, W):
    su = jnp.asarray(_SU_NP)
    idx = jnp.asarray(_IDX_NP)
    p_all = jnp.asarray(_P_NP)
    base = (np.arange(_IPT, dtype=np.int32) // _IPU).reshape(_IPT // _ICH,
                                                             _ICH)
    dst = jnp.asarray(
        base[None] + _UPT * np.arange(16, dtype=np.int32)[:, None, None])
    zeros = jnp.zeros((_UPT, _D), jnp.float32)
    u_rows, b_rows = _sc_gather(user_emb, item_emb, su, idx, dst, zeros)
    u_all = u_rows.reshape(_T, _NU, _D)
    b_all = b_rows.reshape(_T, _NU, _D)
    return pl.pallas_call(
        _dense_body,
        out_shape=jax.ShapeDtypeStruct((_D, _D), jnp.float32),
    )(u_all, b_all, p_all, item_emb[0:1], W)


# speedup vs baseline: 1.0216x; 1.0216x over previous
import functools

import numpy as np
import jax
from jax import lax
import jax.numpy as jnp
from jax.experimental import pallas as pl
from jax.experimental.pallas import tpu as pltpu
from jax.experimental.pallas import tpu_sc as plsc

_NUM_USERS = 10000
_NUM_ITEMS = 50000
_D = 128
_NU = 256
_T = 4
_THRESHOLD = 0.5
_IPU = 16

_NW = 32
_NI = _T * _IPU * _NU
_NUALL = _T * _NU
_IPT = _NI // _NW
_UPT = _NUALL // _NW
_ICH = 128


def _build_consts():
    rng = np.random.default_rng(42)
    user_pos = [np.unique(rng.integers(0, _NUM_ITEMS, _IPU))
                for _ in range(_NUM_USERS)]
    srng = np.random.default_rng(1234)
    su_list, idx_list, pads, p_list = [], [], [], []
    for t in range(_T):
        sampled = np.sort(srng.choice(_NUM_USERS, size=_NU, replace=False))
        items = [user_pos[int(u)] for u in sampled]
        ipad = np.zeros((_NU, _IPU), np.int32)
        for i, its in enumerate(items):
            ipad[i, :len(its)] = its
            for k in range(len(its), _IPU):
                pads.append((t, k, i))
        union = np.unique(np.concatenate(items))
        M = np.zeros((_NU, len(union)), np.float32)
        for i, its in enumerate(items):
            M[i, np.searchsorted(union, its)] = 1.0
        su_list.append(sampled.astype(np.int32))
        idx_list.append(ipad.reshape(-1))
        p_list.append(M @ M.T)
    su_all = np.concatenate(su_list).reshape(1, _T * _NU)
    idx_all = np.concatenate(idx_list).reshape(1, _T * _IPU * _NU)
    p_all = np.stack(p_list).astype(np.float32)
    return su_all, idx_all, pads, p_all


_SU_NP, _IDX_NP, _PADS, _P_NP = _build_consts()


def _sc_gather(user_emb, item_emb, su, idx, dst, zeros):
    mesh = plsc.VectorSubcoreMesh(core_axis_name="c", subcore_axis_name="s")
    n_ch = _IPT // _ICH

    @functools.partial(
        pl.kernel,
        out_type=(jax.ShapeDtypeStruct((_NUALL, _D), jnp.float32),
                  jax.ShapeDtypeStruct((_NUALL, _D), jnp.float32)),
        mesh=mesh,
        scratch_types=[
            pltpu.VMEM((_UPT,), jnp.int32),
            pltpu.VMEM((n_ch, _ICH), jnp.int32),
            pltpu.VMEM((n_ch, _ICH), jnp.int32),
            pltpu.VMEM((_UPT, _D), jnp.float32),
            pltpu.VMEM((_IPT, _D), jnp.float32),
            pltpu.VMEM_SHARED((16 * _UPT, _D), jnp.float32),
            pltpu.SemaphoreType.DMA,
            pltpu.SemaphoreType.DMA,
            pltpu.SemaphoreType.DMA,
            pltpu.SemaphoreType.DMA,
            pltpu.SemaphoreType.DMA,
            pltpu.SemaphoreType.DMA,
        ])
    def gather_kernel(ue_hbm, ie_hbm, su_hbm, ix_hbm, dst_hbm, z_hbm,
                      ou_hbm, ob_hbm,
                      su_v, ix_v, dst_v, ur_v, ir_v, b_sh,
                      s_idx, s_z, s_u, s_i, s_a, s_uo):
        sidx = lax.axis_index("s")
        wid = sidx * 2 + lax.axis_index("c")
        ub = wid * _UPT
        idx_cp = [pltpu.async_copy(su_hbm.at[wid], su_v, s_idx),
                  pltpu.async_copy(ix_hbm.at[wid], ix_v, s_idx),
                  pltpu.async_copy(dst_hbm.at[sidx], dst_v, s_idx)]
        z_cp = pltpu.async_copy(z_hbm, b_sh.at[pl.ds(sidx * _UPT, _UPT)], s_z)
        for cp in idx_cp:
            cp.wait()
        u_cp = pltpu.async_copy(ue_hbm.at[su_v], ur_v, s_u)
        i_cp = [pltpu.async_copy(
            ie_hbm.at[ix_v.at[c]], ir_v.at[pl.ds(c * _ICH, _ICH)], s_i)
            for c in range(n_ch)]
        u_cp.wait()
        uo_cp = pltpu.async_copy(ur_v, ou_hbm.at[pl.ds(ub, _UPT)], s_uo)
        for cp in i_cp:
            cp.wait()
        z_cp.wait()
        a_cp = [pltpu.async_copy(
            ir_v.at[pl.ds(c * _ICH, _ICH)], b_sh.at[dst_v.at[c]], s_a,
            add=True) for c in range(n_ch)]
        for cp in a_cp:
            cp.wait()
        pltpu.sync_copy(b_sh.at[pl.ds(sidx * _UPT, _UPT)],
                        ob_hbm.at[pl.ds(ub, _UPT)])
        uo_cp.wait()

    return gather_kernel(user_emb, item_emb,
                         su.reshape(_NW, _UPT),
                         idx.reshape(_NW, n_ch, _ICH),
                         dst, zeros)


def _dense_body(u_ref, b_ref, p_ref, it0_ref, w_ref, o_ref):
    w = w_ref[...]
    it0 = jnp.broadcast_to(it0_ref[...], (_NU, _D))
    acc = jnp.zeros((_D, _D), jnp.float32)
    for t in range(_T):
        ut = u_ref[t]
        n2 = jnp.sum(ut * ut, axis=1, keepdims=True)
        un = ut / jnp.maximum(jnp.sqrt(n2), 1e-12)
        s = jnp.dot(un, un.T, preferred_element_type=jnp.float32)
        ii = jax.lax.broadcasted_iota(jnp.int32, (_NU, _NU), 0)
        jj = jax.lax.broadcasted_iota(jnp.int32, (_NU, _NU), 1)
        a = jnp.where((s > _THRESHOLD) & (ii != jj), 1.0, 0.0)
        b = b_ref[t]
        rr = jax.lax.broadcasted_iota(jnp.int32, (_NU, _D), 0)
        for (tt, _kk, uu) in _PADS:
            if tt == t:
                b = jnp.where(rr == uu, b - it0, b)
        hu = jnp.dot(a, ut, preferred_element_type=jnp.float32) + b
        pu = jnp.dot(p_ref[t], ut, preferred_element_type=jnp.float32)
        g = (jnp.dot(hu.T, hu, preferred_element_type=jnp.float32)
             + jnp.dot(ut.T, pu, preferred_element_type=jnp.float32))
        wg = jnp.dot(w, g, preferred_element_type=jnp.float32)
        wt = jnp.dot(wg, w.T, preferred_element_type=jnp.float32)
        fro = jnp.sqrt(jnp.sum(wt * wt)) + 1e-8
        acc = acc + wt / fro
    o_ref[...] = acc * (1.0 / _T)


def kernel(user_emb, item_emb, W):
    su = jnp.asarray(_SU_NP)
    idx = jnp.asarray(_IDX_NP)
    p_all = jnp.asarray(_P_NP)
    base = (np.arange(_IPT, dtype=np.int32) // _IPU).reshape(_IPT // _ICH,
                                                             _ICH)
    dst = jnp.asarray(
        base[None] + _UPT * np.arange(16, dtype=np.int32)[:, None, None])
    zeros = jnp.zeros((_UPT, _D), jnp.float32)
    u_rows, b_rows = _sc_gather(user_emb, item_emb, su, idx, dst, zeros)
    u_all = u_rows.reshape(_T, _NU, _D)
    b_all = b_rows.reshape(_T, _NU, _D)
    return pl.pallas_call(
        _dense_body,
        out_shape=jax.ShapeDtypeStruct((_D, _D), jnp.float32),
    )(u_all, b_all, p_all, item_emb[0:1], W)


# speedup vs baseline: 1.0776x; 1.0549x over previous
import functools

import numpy as np
import jax
from jax import lax
import jax.numpy as jnp
from jax.experimental import pallas as pl
from jax.experimental.pallas import tpu as pltpu
from jax.experimental.pallas import tpu_sc as plsc

_NUM_USERS = 10000
_NUM_ITEMS = 50000
_D = 128
_NU = 256
_T = 4
_THRESHOLD = 0.5
_IPU = 16

_NW = 32
_NI = _T * _IPU * _NU
_NUALL = _T * _NU
_IPT = _NI // _NW
_UPT = _NUALL // _NW
_ICH = 128


def _build_consts():
    rng = np.random.default_rng(42)
    user_pos = [np.unique(rng.integers(0, _NUM_ITEMS, _IPU))
                for _ in range(_NUM_USERS)]
    srng = np.random.default_rng(1234)
    su_list, idx_list, pads, p_list = [], [], [], []
    for t in range(_T):
        sampled = np.sort(srng.choice(_NUM_USERS, size=_NU, replace=False))
        items = [user_pos[int(u)] for u in sampled]
        ipad = np.zeros((_NU, _IPU), np.int32)
        for i, its in enumerate(items):
            ipad[i, :len(its)] = its
            for k in range(len(its), _IPU):
                pads.append((t, k, i))
        union = np.unique(np.concatenate(items))
        M = np.zeros((_NU, len(union)), np.float32)
        for i, its in enumerate(items):
            M[i, np.searchsorted(union, its)] = 1.0
        su_list.append(sampled.astype(np.int32))
        idx_list.append(ipad.T.reshape(-1))
        p_list.append(M @ M.T)
    su_all = np.concatenate(su_list).reshape(1, _T * _NU)
    idx_all = np.concatenate(idx_list).reshape(1, _T * _IPU * _NU)
    p_all = np.stack(p_list).astype(np.float32)
    return su_all, idx_all, pads, p_all


_SU_NP, _IDX_NP, _PADS, _P_NP = _build_consts()

_PK_NP = np.zeros((_NW, 1 + _IPT // _ICH, _ICH), np.int32)
_PK_NP[:, 0, :_UPT] = _SU_NP.reshape(_NW, _UPT)
_PK_NP[:, 1:, :] = _IDX_NP.reshape(_NW, _IPT // _ICH, _ICH)


def _sc_gather(user_emb, item_emb, pk):
    mesh = plsc.VectorSubcoreMesh(core_axis_name="c", subcore_axis_name="s")
    n_ch = _IPT // _ICH

    @functools.partial(
        pl.kernel,
        out_type=(jax.ShapeDtypeStruct((_NUALL, _D), jnp.float32),
                  jax.ShapeDtypeStruct((_NI, _D), jnp.float32)),
        mesh=mesh,
        scratch_types=[
            pltpu.VMEM((1 + n_ch, _ICH), jnp.int32),
            pltpu.VMEM((_UPT, _D), jnp.float32),
            pltpu.VMEM((_IPT, _D), jnp.float32),
            pltpu.SemaphoreType.DMA,
            pltpu.SemaphoreType.DMA,
            pltpu.SemaphoreType.DMA,
        ])
    def gather_kernel(ue_hbm, ie_hbm, pk_hbm, ou_hbm, oi_hbm,
                      pk_v, ur_v, ir_v, s_u, s_i, s_o):
        wid = lax.axis_index("s") * 2 + lax.axis_index("c")
        ub = wid * _UPT
        ib = wid * _IPT
        pltpu.sync_copy(pk_hbm.at[wid], pk_v)
        u_cp = pltpu.async_copy(ue_hbm.at[pk_v.at[0, pl.ds(0, _UPT)]],
                                ur_v, s_u)
        i_cp = [pltpu.async_copy(
            ie_hbm.at[pk_v.at[1 + c]], ir_v.at[pl.ds(c * _ICH, _ICH)], s_i)
            for c in range(n_ch)]
        u_cp.wait()
        o_cp = [pltpu.async_copy(ur_v, ou_hbm.at[pl.ds(ub, _UPT)], s_o)]
        for c in range(n_ch):
            i_cp[c].wait()
            o_cp.append(pltpu.async_copy(
                ir_v.at[pl.ds(c * _ICH, _ICH)],
                oi_hbm.at[pl.ds(ib + c * _ICH, _ICH)], s_o))
        for cp in o_cp:
            cp.wait()

    return gather_kernel(user_emb, item_emb, pk)


def _dense_body(u_ref, it_ref, p_ref, w_ref, o_ref):
    w = w_ref[...]
    acc = jnp.zeros((_D, _D), jnp.float32)
    for t in range(_T):
        ut = u_ref[t]
        n2 = jnp.sum(ut * ut, axis=1, keepdims=True)
        un = ut / jnp.maximum(jnp.sqrt(n2), 1e-12)
        s = jnp.dot(un, un.T, preferred_element_type=jnp.float32)
        ii = jax.lax.broadcasted_iota(jnp.int32, (_NU, _NU), 0)
        jj = jax.lax.broadcasted_iota(jnp.int32, (_NU, _NU), 1)
        a = jnp.where((s > _THRESHOLD) & (ii != jj), 1.0, 0.0)
        b = jnp.zeros((_NU, _D), jnp.float32)
        for k in range(_IPU):
            b = b + it_ref[t * _IPU + k]
        rr = jax.lax.broadcasted_iota(jnp.int32, (_NU, _D), 0)
        for (tt, kk, uu) in _PADS:
            if tt == t:
                row = it_ref[tt * _IPU + kk][uu:uu + 1, :]
                b = jnp.where(rr == uu,
                              b - jnp.broadcast_to(row, (_NU, _D)), b)
        hu = jnp.dot(a, ut, preferred_element_type=jnp.float32) + b
        pu = jnp.dot(p_ref[t], ut, preferred_element_type=jnp.float32)
        g = (jnp.dot(hu.T, hu, preferred_element_type=jnp.float32)
             + jnp.dot(ut.T, pu, preferred_element_type=jnp.float32))
        wg = jnp.dot(w, g, preferred_element_type=jnp.float32)
        wt = jnp.dot(wg, w.T, preferred_element_type=jnp.float32)
        fro = jnp.sqrt(jnp.sum(wt * wt)) + 1e-8
        acc = acc + wt / fro
    o_ref[...] = acc * (1.0 / _T)


def kernel(user_emb, item_emb, W):
    p_all = jnp.asarray(_P_NP)
    pk = jnp.asarray(_PK_NP)
    u_rows, it_rows = _sc_gather(user_emb, item_emb, pk)
    u_all = u_rows.reshape(_T, _NU, _D)
    it_all = it_rows.reshape(_T * _IPU, _NU, _D)
    return pl.pallas_call(
        _dense_body,
        out_shape=jax.ShapeDtypeStruct((_D, _D), jnp.float32),
    )(u_all, it_all, p_all, W)
